# Initial kernel scaffold; baseline (speedup 1.0000x reference)
import functools

import jax
import jax.numpy as jnp
from jax import lax
from jax.experimental import pallas as pl
from jax.experimental.pallas import tpu as pltpu
from jax.experimental.pallas import tpu_sc as plsc

N_NODES = 10000
N_EDGES = 320000
D = 128

NC = 2
NS = 16
NW = NC * NS

CHUNK = 128
CPW = 79
EPW = CHUNK * CPW
E_PAD = EPW * NW
RPW = N_NODES // NS


def _matmul_tc(x, W):
    def body(x_ref, w_ref, o_ref):
        o_ref[...] = jnp.dot(x_ref[...], w_ref[...],
                             preferred_element_type=jnp.float32)

    grid = 5
    blk = N_NODES // grid
    return pl.pallas_call(
        body,
        grid=(grid,),
        in_specs=[
            pl.BlockSpec((blk, D), lambda i: (i, 0)),
            pl.BlockSpec((D, D), lambda i: (0, 0)),
        ],
        out_specs=pl.BlockSpec((blk, D), lambda i: (i, 0)),
        out_shape=jax.ShapeDtypeStruct((N_NODES, D), jnp.float32),
    )(x, W)


def _add_tc(partials):
    def body(p_ref, o_ref):
        o_ref[...] = p_ref[0] + p_ref[1]

    grid = 10
    blk = N_NODES // grid
    return pl.pallas_call(
        body,
        grid=(grid,),
        in_specs=[pl.BlockSpec((NC, blk, D), lambda i: (0, i, 0))],
        out_specs=pl.BlockSpec((blk, D), lambda i: (i, 0)),
        out_shape=jax.ShapeDtypeStruct((N_NODES, D), jnp.float32),
    )(partials)


def _spmv_sc(support, row, col, w):
    mesh = plsc.VectorSubcoreMesh(core_axis_name="c", subcore_axis_name="s")

    @functools.partial(
        pl.kernel,
        mesh=mesh,
        out_type=jax.ShapeDtypeStruct((NC, N_NODES, D), jnp.float32),
        scratch_types=[
            pltpu.VMEM((CHUNK,), jnp.int32),
            pltpu.VMEM((CHUNK,), jnp.int32),
            pltpu.VMEM((CHUNK,), jnp.float32),
            pltpu.VMEM((CHUNK, D), jnp.float32),
            pltpu.VMEM_SHARED((N_NODES, D), jnp.float32),
            pltpu.SemaphoreType.DMA,
        ],
    )
    def k(support_hbm, row_hbm, col_hbm, w_hbm, out_hbm,
          colv, rowv, wv, rows, acc, sem):
        c = lax.axis_index("c")
        s = lax.axis_index("s")
        wid = s * NC + c

        def zrow(i, _):
            zero = jnp.zeros((16,), jnp.float32)
            for j in range(D // 16):
                rows[i, pl.ds(j * 16, 16)] = zero
            return 0
        lax.fori_loop(0, CHUNK, zrow, 0)
        rbase = s * RPW
        for q in range(4):
            pltpu.sync_copy(rows.at[...],
                            acc.at[pl.ds(rbase + q * CHUNK, CHUNK)])
        pltpu.sync_copy(rows.at[pl.ds(0, RPW - 4 * CHUNK)],
                        acc.at[pl.ds(rbase + 4 * CHUNK, RPW - 4 * CHUNK)])
        plsc.subcore_barrier()

        ebase = wid * EPW

        def chunk_body(kk, _):
            b = ebase + kk * CHUNK
            pltpu.sync_copy(col_hbm.at[pl.ds(b, CHUNK)], colv)
            pltpu.sync_copy(row_hbm.at[pl.ds(b, CHUNK)], rowv)
            pltpu.sync_copy(w_hbm.at[pl.ds(b, CHUNK)], wv)
            pltpu.async_copy(support_hbm.at[colv], rows, sem).wait()

            def edge_body(e, _):
                eidx = jnp.full((16,), 0, jnp.int32) + e
                wb = plsc.load_gather(wv, [eidx])
                for j in range(D // 16):
                    v = rows[e, pl.ds(j * 16, 16)]
                    rows[e, pl.ds(j * 16, 16)] = v * wb
                return 0
            lax.fori_loop(0, CHUNK, edge_body, 0)

            pltpu.sync_copy(rows, acc.at[rowv], add=True)
            return 0
        lax.fori_loop(0, CPW, chunk_body, 0)

        plsc.subcore_barrier()
        for q in range(4):
            pltpu.sync_copy(acc.at[pl.ds(rbase + q * CHUNK, CHUNK)],
                            out_hbm.at[c, pl.ds(rbase + q * CHUNK, CHUNK)])
        pltpu.sync_copy(acc.at[pl.ds(rbase + 4 * CHUNK, RPW - 4 * CHUNK)],
                        out_hbm.at[c, pl.ds(rbase + 4 * CHUNK, RPW - 4 * CHUNK)])

    return k(support, row, col, w)


def kernel(x, edge_index, edge_weight, W):
    support = _matmul_tc(x, W)

    row = edge_index[0].astype(jnp.int32)
    col = edge_index[1].astype(jnp.int32)
    w = edge_weight.astype(jnp.float32)

    pad = E_PAD - N_EDGES
    zi = jnp.zeros((pad,), jnp.int32)
    row = jnp.concatenate([row, zi])
    col = jnp.concatenate([col, zi])
    w = jnp.concatenate([w, jnp.zeros((pad,), jnp.float32)])

    partials = _spmv_sc(support, row, col, w)
    return _add_tc(partials)


# speedup vs baseline: 3.9804x; 3.9804x over previous
import functools

import jax
import jax.numpy as jnp
from jax import lax
from jax.experimental import pallas as pl
from jax.experimental.pallas import tpu as pltpu
from jax.experimental.pallas import tpu_sc as plsc

N_NODES = 10000
N_EDGES = 320000
D = 128

NC = 2
NS = 16
NW = NC * NS

CHUNK = 128
CPW = 79
EPW = CHUNK * CPW
E_PAD = EPW * NW
RPW = 624


def _matmul_tc(x, W):
    def body(x_ref, w_ref, o_ref):
        o_ref[...] = jnp.dot(x_ref[...], w_ref[...],
                             preferred_element_type=jnp.float32)

    grid = 5
    blk = N_NODES // grid
    return pl.pallas_call(
        body,
        grid=(grid,),
        in_specs=[
            pl.BlockSpec((blk, D), lambda i: (i, 0)),
            pl.BlockSpec((D, D), lambda i: (0, 0)),
        ],
        out_specs=pl.BlockSpec((blk, D), lambda i: (i, 0)),
        out_shape=jax.ShapeDtypeStruct((N_NODES, D), jnp.float32),
    )(x, W)


def _add_tc(partials):
    def body(p_ref, o_ref):
        o_ref[...] = p_ref[0] + p_ref[1]

    grid = 10
    blk = N_NODES // grid
    return pl.pallas_call(
        body,
        grid=(grid,),
        in_specs=[pl.BlockSpec((NC, blk, D), lambda i: (0, i, 0))],
        out_specs=pl.BlockSpec((blk, D), lambda i: (i, 0)),
        out_shape=jax.ShapeDtypeStruct((N_NODES, D), jnp.float32),
    )(partials)


def _spmv_sc(support, row, col, w):
    mesh = plsc.VectorSubcoreMesh(core_axis_name="c", subcore_axis_name="s")

    @functools.partial(
        pl.kernel,
        mesh=mesh,
        out_type=jax.ShapeDtypeStruct((NC, N_NODES, D), jnp.float32),
        scratch_types=[
            pltpu.VMEM((CHUNK,), jnp.int32),
            pltpu.VMEM((CHUNK,), jnp.int32),
            pltpu.VMEM((CHUNK,), jnp.float32),
            pltpu.VMEM((CHUNK, D), jnp.float32),
            pltpu.VMEM_SHARED((N_NODES, D), jnp.float32),
            pltpu.SemaphoreType.DMA,
        ],
    )
    def k(support_hbm, row_hbm, col_hbm, w_hbm, out_hbm,
          colv, rowv, wv, rows, acc, sem):
        c = lax.axis_index("c")
        s = lax.axis_index("s")
        wid = s * NC + c

        def zrow(i, _):
            zero = jnp.zeros((16,), jnp.float32)
            for j in range(D // 16):
                rows[i, pl.ds(j * 16, 16)] = zero
            return 0
        lax.fori_loop(0, CHUNK, zrow, 0)
        rbase = s * RPW
        for q in range(4):
            pltpu.sync_copy(rows.at[...],
                            acc.at[pl.ds(rbase + q * CHUNK, CHUNK)])
        pltpu.sync_copy(rows.at[pl.ds(0, RPW - 4 * CHUNK)],
                        acc.at[pl.ds(rbase + 4 * CHUNK, RPW - 4 * CHUNK)])

        @pl.when(s == NS - 1)
        def _():
            pltpu.sync_copy(rows.at[pl.ds(0, N_NODES - NS * RPW)],
                            acc.at[pl.ds(NS * RPW, N_NODES - NS * RPW)])
        plsc.subcore_barrier()

        ebase = wid * EPW

        def chunk_body(kk, _):
            b = ebase + kk * CHUNK
            pltpu.sync_copy(col_hbm.at[pl.ds(b, CHUNK)], colv)
            pltpu.sync_copy(row_hbm.at[pl.ds(b, CHUNK)], rowv)
            pltpu.sync_copy(w_hbm.at[pl.ds(b, CHUNK)], wv)
            pltpu.async_copy(support_hbm.at[colv], rows, sem).wait()

            def group_body(g, _):
                wvec = wv[pl.ds(g * 16, 16)]
                for i in range(16):
                    e = g * 16 + i
                    wb = wvec[i]
                    for j in range(D // 16):
                        v = rows[e, pl.ds(j * 16, 16)]
                        rows[e, pl.ds(j * 16, 16)] = v * wb
                return 0
            lax.fori_loop(0, CHUNK // 16, group_body, 0)

            pltpu.sync_copy(rows, acc.at[rowv], add=True)
            return 0
        lax.fori_loop(0, CPW, chunk_body, 0)

        plsc.subcore_barrier()
        for q in range(4):
            pltpu.sync_copy(acc.at[pl.ds(rbase + q * CHUNK, CHUNK)],
                            out_hbm.at[c, pl.ds(rbase + q * CHUNK, CHUNK)])
        pltpu.sync_copy(acc.at[pl.ds(rbase + 4 * CHUNK, RPW - 4 * CHUNK)],
                        out_hbm.at[c, pl.ds(rbase + 4 * CHUNK, RPW - 4 * CHUNK)])

        @pl.when(s == NS - 1)
        def _():
            pltpu.sync_copy(acc.at[pl.ds(NS * RPW, N_NODES - NS * RPW)],
                            out_hbm.at[c, pl.ds(NS * RPW, N_NODES - NS * RPW)])

    return k(support, row, col, w)


def kernel(x, edge_index, edge_weight, W):
    support = _matmul_tc(x, W)

    row = edge_index[0].astype(jnp.int32)
    col = edge_index[1].astype(jnp.int32)
    w = edge_weight.astype(jnp.float32)

    pad = E_PAD - N_EDGES
    zi = jnp.zeros((pad,), jnp.int32)
    row = jnp.concatenate([row, zi])
    col = jnp.concatenate([col, zi])
    w = jnp.concatenate([w, jnp.zeros((pad,), jnp.float32)])

    partials = _spmv_sc(support, row, col, w)
    return _add_tc(partials)
